# baseline (speedup 1.0000x reference)
import functools

import numpy as np
import jax
import jax.numpy as jnp
from jax import lax
from jax.experimental import pallas as pl
from jax.experimental.pallas import tpu as pltpu
from jax.experimental.pallas import tpu_sc as plsc

B, X, C_IN = 8, 16, 32
K, S, F = 3, 2, 64
OX = (X - K) // S + 1
NPOS = OX * OX * OX
CK = K * K * K * C_IN
NTAP = K * K * K

NW = 32
NCH = 3
XPARTS = ((0, 4), (4, 3))


def _pad_rows(nrows):
    q = NW * 8 * NCH * 27
    return -(-nrows // q) * q


def _sc_row_ids(x0, nx):
    nrows = nx * OX * OX * B * NTAP
    pad = _pad_rows(nrows)
    idx = np.zeros((pad,), np.int32)
    r = 0
    for p in range(nx * OX * OX):
        x, y, z = x0 + p // (OX * OX), (p // OX) % OX, p % OX
        for b in range(B):
            for i in range(K):
                for j in range(K):
                    for l in range(K):
                        idx[r] = ((b * X + (S * x + i)) * X
                                  + (S * y + j)) * X + (S * z + l)
                        r += 1
    return idx.reshape(NW * NCH, 1, pad // (NW * NCH))


_IDX = tuple(_sc_row_ids(x0, nx) for x0, nx in XPARTS)


def _make_sc_gather(rows_pad):
    rpw = rows_pad // NW
    ch = rpw // NCH

    def body(tbl_hbm, stbl_hbm, idx_hbm, p_hbm, s_hbm,
             idx_v, rows_v, srows_v, sem_p, sem_s):
        wid = lax.axis_index("s") * 2 + lax.axis_index("c")
        for c in range(NCH):
            base = wid * rpw + c * ch
            pltpu.sync_copy(idx_hbm.at[wid * NCH + c, 0], idx_v)
            cp_p = pltpu.async_copy(tbl_hbm.at[idx_v], rows_v, sem_p)
            cp_s = pltpu.async_copy(stbl_hbm.at[idx_v], srows_v, sem_s)
            cp_p.wait()
            cp_s.wait()
            pltpu.sync_copy(rows_v, p_hbm.at[pl.ds(base, ch)])
            pltpu.sync_copy(srows_v, s_hbm.at[pl.ds(base, ch)])

    mesh = plsc.VectorSubcoreMesh(core_axis_name="c", subcore_axis_name="s")
    return functools.partial(
        pl.kernel,
        out_type=[jax.ShapeDtypeStruct((rows_pad, C_IN), jnp.float32)] * 2,
        mesh=mesh,
        scratch_types=[
            pltpu.VMEM((ch,), jnp.int32),
            pltpu.VMEM((ch, C_IN), jnp.float32),
            pltpu.VMEM((ch, C_IN), jnp.float32),
            pltpu.SemaphoreType.DMA,
            pltpu.SemaphoreType.DMA,
        ],
        compiler_params=pltpu.CompilerParams(use_tc_tiling_on_sc=False),
    )(body)


def _mm_body(p_ref, sp_ref, loc_ref, rho_ref, eps_ref, b_ref, so_ref, o_ref):
    for z in range(OX):
        p = p_ref[pl.ds(z * B, B)]
        ps = p * sp_ref[pl.ds(z * B, B)]
        loc = loc_ref[0, 0, z].reshape(CK, F)
        w2 = (jax.nn.softplus(rho_ref[0, 0, z].reshape(CK, F))
              * eps_ref[0, 0, z].reshape(CK, F))
        m = jnp.dot(p, loc, preferred_element_type=jnp.float32)
        pert = jnp.dot(ps, w2, preferred_element_type=jnp.float32)
        o_ref[z] = m + b_ref[0, 0, z][None, :] + pert * so_ref[:, 0, 0, z, :]


def _mm_part(x0, nx, pv, sv, kernel_loc, kernel_rho, eps, bias, sign_out):
    pspec = pl.BlockSpec((OX * B, CK), lambda i: (i, 0))
    wspec = pl.BlockSpec((1, 1, OX, K, K, K, C_IN, F),
                         lambda i: (x0 + i // OX, i % OX, 0, 0, 0, 0, 0, 0))
    return pl.pallas_call(
        _mm_body,
        grid=(nx * OX,),
        in_specs=[
            pspec, pspec, wspec, wspec, wspec,
            pl.BlockSpec((1, 1, OX, F),
                         lambda i: (x0 + i // OX, i % OX, 0, 0)),
            pl.BlockSpec((B, 1, 1, OX, F),
                         lambda i: (0, x0 + i // OX, i % OX, 0, 0)),
        ],
        out_specs=pl.BlockSpec((OX, B, F), lambda i: (i, 0, 0)),
        out_shape=jax.ShapeDtypeStruct((nx * OX * OX, B, F), jnp.float32),
    )(pv, sv, kernel_loc, kernel_rho, eps, bias, sign_out)


def kernel(inputs, kernel_loc, kernel_rho, bias, eps, sign_in, sign_out):
    tbl = inputs.reshape(B * X * X * X, C_IN)
    stbl = sign_in.reshape(B * X * X * X, C_IN)

    gathered = []
    for part in range(len(XPARTS)):
        rows_pad = _IDX[part].size
        pout, sout = _make_sc_gather(rows_pad)(tbl, stbl,
                                               jnp.asarray(_IDX[part]))
        gathered.append((pout.reshape(rows_pad * C_IN // CK, CK),
                         sout.reshape(rows_pad * C_IN // CK, CK)))

    outs = []
    for part, (x0, nx) in enumerate(XPARTS):
        pv, sv = gathered[part]
        outs.append(_mm_part(x0, nx, pv, sv, kernel_loc, kernel_rho, eps,
                             bias, sign_out))

    out = jnp.concatenate(outs, axis=0)
    return out.reshape(OX, OX, OX, B, F).transpose(3, 0, 1, 2, 4)


# speedup vs baseline: 3.0760x; 3.0760x over previous
import jax
import jax.numpy as jnp
from jax.experimental import pallas as pl

B, X, C_IN = 8, 16, 32
K, S, F = 3, 2, 64
OX = (X - K) // S + 1
NPOS = OX * OX * OX
CK = K * K * K * C_IN


def _body(x_ref, s_ref, loc_ref, rho_ref, eps_ref, b_ref, so_ref, o_ref):
    i = pl.program_id(0)
    x = i // OX
    y = i % OX
    win = x_ref[:, pl.ds(2 * x, K), pl.ds(2 * y, K), :, :]
    sw = win * s_ref[:, pl.ds(2 * x, K), pl.ds(2 * y, K), :, :]
    for z in range(OX):
        p = win[:, :, :, 2 * z:2 * z + K, :].reshape(B, CK)
        ps = sw[:, :, :, 2 * z:2 * z + K, :].reshape(B, CK)
        loc = loc_ref[0, 0, z].reshape(CK, F)
        w2 = (jax.nn.softplus(rho_ref[0, 0, z].reshape(CK, F))
              * eps_ref[0, 0, z].reshape(CK, F))
        m = jnp.dot(p, loc, preferred_element_type=jnp.float32)
        pert = jnp.dot(ps, w2, preferred_element_type=jnp.float32)
        o_ref[:, 0, 0, z, :] = (m + b_ref[0, 0, z][None, :]
                                + pert * so_ref[:, 0, 0, z, :])


def kernel(inputs, kernel_loc, kernel_rho, bias, eps, sign_in, sign_out):
    full_in = pl.BlockSpec((B, X, X, X, C_IN), lambda i: (0, 0, 0, 0, 0))
    wspec = pl.BlockSpec((1, 1, OX, K, K, K, C_IN, F),
                         lambda i: (i // OX, i % OX, 0, 0, 0, 0, 0, 0))
    out = pl.pallas_call(
        _body,
        grid=(OX * OX,),
        in_specs=[
            full_in, full_in, wspec, wspec, wspec,
            pl.BlockSpec((1, 1, OX, F), lambda i: (i // OX, i % OX, 0, 0)),
            pl.BlockSpec((B, 1, 1, OX, F), lambda i: (0, i // OX, i % OX, 0, 0)),
        ],
        out_specs=pl.BlockSpec((B, 1, 1, OX, F),
                               lambda i: (0, i // OX, i % OX, 0, 0)),
        out_shape=jax.ShapeDtypeStruct((B, OX, OX, OX, F), jnp.float32),
    )(inputs, sign_in, kernel_loc, kernel_rho, eps, bias, sign_out)

    return out


# speedup vs baseline: 3.1351x; 1.0192x over previous
import jax
import jax.numpy as jnp
from jax.experimental import pallas as pl

B, X, C_IN = 8, 16, 32
K, S, F = 3, 2, 64
OX = (X - K) // S + 1
NPOS = OX * OX * OX
CK = K * K * K * C_IN


def _body(x_ref, s_ref, loc_ref, rho_ref, eps_ref, b_ref, so_ref, o_ref):
    i = pl.program_id(0)
    x = i // OX
    y = i % OX
    win = x_ref[:, pl.ds(2 * x, K), pl.ds(2 * y, K), :, :]
    sw = win * s_ref[:, pl.ds(2 * x, K), pl.ds(2 * y, K), :, :]
    for z in range(OX):
        p = win[:, :, :, 2 * z:2 * z + K, :].reshape(B, CK)
        ps = sw[:, :, :, 2 * z:2 * z + K, :].reshape(B, CK)
        loc = loc_ref[0, 0, z].reshape(CK, F)
        w2 = (jax.nn.softplus(rho_ref[0, 0, z].reshape(CK, F))
              * eps_ref[0, 0, z].reshape(CK, F))
        m = jnp.dot(p, loc, preferred_element_type=jnp.float32)
        pert = jnp.dot(ps, w2, preferred_element_type=jnp.float32)
        o_ref[z] = m + b_ref[0, 0, z][None, :] + pert * so_ref[:, 0, 0, z, :]


def kernel(inputs, kernel_loc, kernel_rho, bias, eps, sign_in, sign_out):
    full_in = pl.BlockSpec((B, X, X, X, C_IN), lambda i: (0, 0, 0, 0, 0))
    wspec = pl.BlockSpec((1, 1, OX, K, K, K, C_IN, F),
                         lambda i: (i // OX, i % OX, 0, 0, 0, 0, 0, 0))
    out = pl.pallas_call(
        _body,
        grid=(OX * OX,),
        in_specs=[
            full_in, full_in, wspec, wspec, wspec,
            pl.BlockSpec((1, 1, OX, F), lambda i: (i // OX, i % OX, 0, 0)),
            pl.BlockSpec((B, 1, 1, OX, F), lambda i: (0, i // OX, i % OX, 0, 0)),
        ],
        out_specs=pl.BlockSpec((OX, B, F), lambda i: (i, 0, 0)),
        out_shape=jax.ShapeDtypeStruct((NPOS, B, F), jnp.float32),
    )(inputs, sign_in, kernel_loc, kernel_rho, eps, bias, sign_out)

    return out.reshape(OX, OX, OX, B, F).transpose(3, 0, 1, 2, 4)
